# baseline (device time: 53336 ns/iter reference)
import jax
import jax.numpy as jnp
from jax import lax
from jax.experimental import pallas as pl
from jax.experimental.pallas import tpu as pltpu

N_DEV = 4
K_CHUNK = 1024


def kernel(x, w_mat):
    m_per, k_dim = x.shape
    _, n_dim = w_mat.shape
    n_per = n_dim // N_DEV
    m_half = m_per // 2
    m_q = m_per // 4

    pieces = {
        1: [(0, m_q), (m_q, m_q), (m_half, m_half)],
        2: [(0, m_half), (m_half, m_half)],
        3: [(0, m_half), (m_half, m_half)],
    }
    slot_of = {}
    for k in range(1, N_DEV):
        for p, _ in enumerate(pieces[k]):
            slot_of[(k, p)] = len(slot_of)
    n_slots = len(slot_of)

    x_pieces = [(0, m_q), (m_q, m_q), (m_half, m_half)]
    def x_deps(row0, rows):
        deps = []
        for i, (r0, rn) in enumerate(x_pieces):
            if r0 < row0 + rows and row0 < r0 + rn:
                deps.append(i)
        return deps

    def body(
        x_hbm,
        w_hbm,
        out_ref,
        xbuf,
        xsems,
        wbuf,
        wsems,
        send_buf,
        recv_buf,
        send_sems,
        recv_sems,
    ):
        me = lax.axis_index("i")

        def w_dma(dest, slot):
            return pltpu.make_async_copy(
                w_hbm.at[:, pl.ds(dest * n_per, n_per)],
                wbuf.at[slot],
                wsems.at[slot],
            )

        def x_dma(i):
            r0, rn = x_pieces[i]
            return pltpu.make_async_copy(
                x_hbm.at[pl.ds(r0, rn), :],
                xbuf.at[pl.ds(r0, rn), :],
                xsems.at[i],
            )

        def gemm_relu(slot, row0, rows):
            acc = jnp.zeros((rows, n_per), jnp.float32)
            for c in range(0, k_dim, K_CHUNK):
                acc += jnp.dot(
                    xbuf[row0 : row0 + rows, c : c + K_CHUNK],
                    wbuf[slot, c : c + K_CHUNK, :],
                    preferred_element_type=jnp.float32,
                )
            return jnp.maximum(acc, 0.0)

        w_dma((me + 1) % N_DEV, 0).start()
        for i in range(len(x_pieces)):
            x_dma(i).start()

        barrier_sem = pltpu.get_barrier_semaphore()
        for k in range(1, N_DEV):
            pl.semaphore_signal(
                barrier_sem,
                inc=1,
                device_id=((me + k) % N_DEV,),
                device_id_type=pl.DeviceIdType.MESH,
            )
        pl.semaphore_wait(barrier_sem, N_DEV - 1)

        rdmas = {}
        x_waited = set()
        for k in range(1, N_DEV):
            dest = (me + k) % N_DEV
            slot = (k - 1) % 2
            w_dma(dest, slot).wait()
            w_dma((me + k + 1) % N_DEV, k % 2).start()
            for p, (row0, rows) in enumerate(pieces[k]):
                for dep in x_deps(row0, rows):
                    if dep not in x_waited:
                        x_dma(dep).wait()
                        x_waited.add(dep)
                send_buf[k - 1, row0 : row0 + rows, :] = gemm_relu(
                    slot, row0, rows
                ).astype(jnp.bfloat16)
                s = slot_of[(k, p)]
                rdma = pltpu.make_async_remote_copy(
                    src_ref=send_buf.at[k - 1, pl.ds(row0, rows)],
                    dst_ref=recv_buf.at[k - 1, pl.ds(row0, rows)],
                    send_sem=send_sems.at[s],
                    recv_sem=recv_sems.at[s],
                    device_id=(dest,),
                    device_id_type=pl.DeviceIdType.MESH,
                )
                rdma.start()
                rdmas[(k, p)] = rdma

        w_dma(me, 1).wait()
        out_ref[pl.ds(me * m_per, m_per), :] = gemm_relu(1, 0, m_per)

        for k in range(1, N_DEV):
            origin = (me - k) % N_DEV
            for p, (row0, rows) in enumerate(pieces[k]):
                rdmas[(k, p)].wait_recv()
                out_ref[pl.ds(origin * m_per + row0, rows), :] = recv_buf[
                    k - 1, row0 : row0 + rows, :
                ].astype(jnp.float32)

        for r in rdmas.values():
            r.wait_send()

    return pl.pallas_call(
        body,
        out_shape=jax.ShapeDtypeStruct((N_DEV * m_per, n_per), jnp.float32),
        in_specs=[
            pl.BlockSpec(memory_space=pltpu.MemorySpace.HBM),
            pl.BlockSpec(memory_space=pltpu.MemorySpace.HBM),
        ],
        out_specs=pl.BlockSpec(memory_space=pltpu.VMEM),
        scratch_shapes=[
            pltpu.VMEM((m_per, k_dim), jnp.float32),
            pltpu.SemaphoreType.DMA((len(x_pieces),)),
            pltpu.VMEM((2, k_dim, n_per), jnp.float32),
            pltpu.SemaphoreType.DMA((2,)),
            pltpu.VMEM((N_DEV - 1, m_per, n_per), jnp.bfloat16),
            pltpu.VMEM((N_DEV - 1, m_per, n_per), jnp.bfloat16),
            pltpu.SemaphoreType.DMA((n_slots,)),
            pltpu.SemaphoreType.DMA((n_slots,)),
        ],
        compiler_params=pltpu.CompilerParams(
            collective_id=0, vmem_limit_bytes=63 * 1024 * 1024
        ),
    )(x, w_mat)


# device time: 51130 ns/iter; 1.0431x vs baseline; 1.0431x over previous
import jax
import jax.numpy as jnp
from jax import lax
from jax.experimental import pallas as pl
from jax.experimental.pallas import tpu as pltpu

N_DEV = 4
K_CHUNK = 1024


def kernel(x, w_mat):
    m_per, k_dim = x.shape
    _, n_dim = w_mat.shape
    n_per = n_dim // N_DEV
    m_half = m_per // 2
    m_q = m_per // 4

    pieces = {
        1: [(0, m_q), (m_q, m_q), (m_half, m_half)],
        2: [(0, m_half), (m_half, m_half)],
        3: [(0, m_half), (m_half, m_half)],
    }
    slot_of = {}
    for k in range(1, N_DEV):
        for p, _ in enumerate(pieces[k]):
            slot_of[(k, p)] = len(slot_of)
    n_slots = len(slot_of)

    x_pieces = [(0, m_q), (m_q, m_q), (m_half, m_half)]
    def x_deps(row0, rows):
        deps = []
        for i, (r0, rn) in enumerate(x_pieces):
            if r0 < row0 + rows and row0 < r0 + rn:
                deps.append(i)
        return deps

    def body(
        x_hbm,
        w_hbm,
        out_hbm,
        xbuf,
        xsems,
        wbuf,
        wsems,
        vout,
        outsems,
        send_buf,
        recv_buf,
        send_sems,
        recv_sems,
    ):
        me = lax.axis_index("i")

        def w_dma(dest, slot):
            return pltpu.make_async_copy(
                w_hbm.at[:, pl.ds(dest * n_per, n_per)],
                wbuf.at[slot],
                wsems.at[slot],
            )

        def x_dma(i):
            r0, rn = x_pieces[i]
            return pltpu.make_async_copy(
                x_hbm.at[pl.ds(r0, rn), :],
                xbuf.at[pl.ds(r0, rn), :],
                xsems.at[i],
            )

        def gemm_relu(slot, row0, rows):
            acc = jnp.zeros((rows, n_per), jnp.float32)
            for c in range(0, k_dim, K_CHUNK):
                acc += jnp.dot(
                    xbuf[row0 : row0 + rows, c : c + K_CHUNK],
                    wbuf[slot, c : c + K_CHUNK, :],
                    preferred_element_type=jnp.float32,
                )
            return jnp.maximum(acc, 0.0)

        w_dma((me + 1) % N_DEV, 0).start()
        for i in range(len(x_pieces)):
            x_dma(i).start()

        barrier_sem = pltpu.get_barrier_semaphore()
        for k in range(1, N_DEV):
            pl.semaphore_signal(
                barrier_sem,
                inc=1,
                device_id=((me + k) % N_DEV,),
                device_id_type=pl.DeviceIdType.MESH,
            )
        pl.semaphore_wait(barrier_sem, N_DEV - 1)

        rdmas = {}
        out_dmas = []
        x_waited = set()
        for k in range(1, N_DEV):
            dest = (me + k) % N_DEV
            slot = (k - 1) % 2
            w_dma(dest, slot).wait()
            w_dma((me + k + 1) % N_DEV, k % 2).start()
            for p, (row0, rows) in enumerate(pieces[k]):
                for dep in x_deps(row0, rows):
                    if dep not in x_waited:
                        x_dma(dep).wait()
                        x_waited.add(dep)
                send_buf[k - 1, row0 : row0 + rows, :] = gemm_relu(
                    slot, row0, rows
                ).astype(jnp.bfloat16)
                s = slot_of[(k, p)]
                rdma = pltpu.make_async_remote_copy(
                    src_ref=send_buf.at[k - 1, pl.ds(row0, rows)],
                    dst_ref=recv_buf.at[k - 1, pl.ds(row0, rows)],
                    send_sem=send_sems.at[s],
                    recv_sem=recv_sems.at[s],
                    device_id=(dest,),
                    device_id_type=pl.DeviceIdType.MESH,
                )
                rdma.start()
                rdmas[(k, p)] = rdma

        w_dma(me, 1).wait()
        vout[pl.ds(me * m_per, m_per), :] = gemm_relu(1, 0, m_per)
        dma = pltpu.make_async_copy(
            vout.at[pl.ds(me * m_per, m_per), :],
            out_hbm.at[pl.ds(me * m_per, m_per), :],
            outsems.at[n_slots],
        )
        dma.start()
        out_dmas.append(dma)

        for k in range(1, N_DEV):
            origin = (me - k) % N_DEV
            for p, (row0, rows) in enumerate(pieces[k]):
                rdmas[(k, p)].wait_recv()
                vout[pl.ds(origin * m_per + row0, rows), :] = recv_buf[
                    k - 1, row0 : row0 + rows, :
                ].astype(jnp.float32)
                dma = pltpu.make_async_copy(
                    vout.at[pl.ds(origin * m_per + row0, rows), :],
                    out_hbm.at[pl.ds(origin * m_per + row0, rows), :],
                    outsems.at[slot_of[(k, p)]],
                )
                dma.start()
                out_dmas.append(dma)

        for d in out_dmas:
            d.wait()
        for r in rdmas.values():
            r.wait_send()

    return pl.pallas_call(
        body,
        out_shape=jax.ShapeDtypeStruct((N_DEV * m_per, n_per), jnp.float32),
        in_specs=[
            pl.BlockSpec(memory_space=pltpu.MemorySpace.HBM),
            pl.BlockSpec(memory_space=pltpu.MemorySpace.HBM),
        ],
        out_specs=pl.BlockSpec(memory_space=pltpu.MemorySpace.HBM),
        scratch_shapes=[
            pltpu.VMEM((m_per, k_dim), jnp.float32),
            pltpu.SemaphoreType.DMA((len(x_pieces),)),
            pltpu.VMEM((2, k_dim, n_per), jnp.float32),
            pltpu.SemaphoreType.DMA((2,)),
            pltpu.VMEM((N_DEV * m_per, n_per), jnp.float32),
            pltpu.SemaphoreType.DMA((n_slots + 1,)),
            pltpu.VMEM((N_DEV - 1, m_per, n_per), jnp.bfloat16),
            pltpu.VMEM((N_DEV - 1, m_per, n_per), jnp.bfloat16),
            pltpu.SemaphoreType.DMA((n_slots,)),
            pltpu.SemaphoreType.DMA((n_slots,)),
        ],
        compiler_params=pltpu.CompilerParams(
            collective_id=0, vmem_limit_bytes=63 * 1024 * 1024
        ),
    )(x, w_mat)
